# Initial kernel scaffold; baseline (speedup 1.0000x reference)
import jax
import jax.numpy as jnp
from jax.experimental import pallas as pl


def kernel(entity_emb, edge_index, edge_type, W, b, gamma, beta):
    raise NotImplementedError("write your pallas kernel here")


# speedup vs baseline: 13.7254x; 13.7254x over previous
import functools

import jax
import jax.numpy as jnp
from jax import lax
from jax.experimental import pallas as pl
from jax.experimental.pallas import tpu as pltpu
from jax.experimental.pallas import tpu_sc as plsc

N = 10000
D = 128
R = 8
E = 320000

CHUNK = 128
CPW = 79
NW = 32
EPAD = NW * CPW * CHUNK
NPAD = 10112
RPT = NPAD // 16
DUMMY = N


def _transform_body(emb_ref, w_ref, b_ref, t_ref):
    x = emb_ref[...]
    w = w_ref[0]
    y = lax.dot_general(x, w, (((1,), (1,)), ((), ())),
                        preferred_element_type=jnp.float32)
    t_ref[0] = y + b_ref[0]


def _transform(emb, W, b):
    BN = 1000
    out = pl.pallas_call(
        _transform_body,
        grid=(R, N // BN),
        in_specs=[
            pl.BlockSpec((BN, D), lambda r, i: (i, 0)),
            pl.BlockSpec((1, D, D), lambda r, i: (r, 0, 0)),
            pl.BlockSpec((1, 1, D), lambda r, i: (r, 0, 0)),
        ],
        out_specs=pl.BlockSpec((1, BN, D), lambda r, i: (r, i, 0)),
        out_shape=jax.ShapeDtypeStruct((R, N, D), jnp.float32),
    )(emb, W, b.reshape(R, 1, D))
    return out.reshape(R * N, D)


def _gidx_body(et_ref, src_ref, g_ref):
    g_ref[...] = et_ref[...] * N + src_ref[...]


def _make_gidx(et_p, src_p):
    rows = EPAD // 128
    BR = rows // 4
    out = pl.pallas_call(
        _gidx_body,
        grid=(4,),
        in_specs=[
            pl.BlockSpec((BR, 128), lambda i: (i, 0)),
            pl.BlockSpec((BR, 128), lambda i: (i, 0)),
        ],
        out_specs=pl.BlockSpec((BR, 128), lambda i: (i, 0)),
        out_shape=jax.ShapeDtypeStruct((rows, 128), jnp.int32),
    )(et_p.reshape(rows, 128), src_p.reshape(rows, 128))
    return out.reshape(EPAD)


@functools.cache
def _sc_scatter_kernel():
    mesh = plsc.VectorSubcoreMesh(core_axis_name="c", subcore_axis_name="s")
    return functools.partial(
        pl.kernel,
        mesh=mesh,
        out_type=jax.ShapeDtypeStruct((2, NPAD, D), jnp.float32),
        scratch_types=[
            pltpu.VMEM((2, CHUNK), jnp.int32),
            pltpu.VMEM((2, CHUNK), jnp.int32),
            pltpu.VMEM((2, CHUNK, D), jnp.float32),
            pltpu.VMEM_SHARED((NPAD, D), jnp.float32),
            pltpu.SemaphoreType.DMA,
        ],
    )(_sc_scatter_body)


def _sc_scatter_body(gidx_hbm, didx_hbm, t_hbm, out_hbm, gbuf, dbuf, rows, acc, sem):
    c = lax.axis_index("c")
    s = lax.axis_index("s")
    wid = s * 2 + c
    row0 = s * RPT

    zero16 = jnp.zeros((16,), jnp.float32)

    def zbody(i, carry):
        for j in range(8):
            rows[0, i, pl.ds(j * 16, 16)] = zero16
        return carry

    lax.fori_loop(0, CHUNK, zbody, 0)
    for t in range(4):
        pltpu.sync_copy(rows.at[0], acc.at[pl.ds(row0 + t * CHUNK, CHUNK)])
    pltpu.sync_copy(rows.at[0, pl.ds(0, RPT - 4 * CHUNK)],
                    acc.at[pl.ds(row0 + 4 * CHUNK, RPT - 4 * CHUNK)])
    plsc.subcore_barrier()

    base = wid * (CPW * CHUNK)

    def body(j, carry):
        off = base + j * CHUNK
        pltpu.sync_copy(gidx_hbm.at[pl.ds(off, CHUNK)], gbuf.at[0])
        pltpu.sync_copy(didx_hbm.at[pl.ds(off, CHUNK)], dbuf.at[0])
        pltpu.async_copy(t_hbm.at[gbuf.at[0]], rows.at[0], sem).wait()
        pltpu.sync_copy(rows.at[0], acc.at[dbuf.at[0]], add=True)
        return carry

    lax.fori_loop(0, CPW, body, 0)
    plsc.subcore_barrier()

    pltpu.sync_copy(acc.at[pl.ds(row0, RPT)],
                    out_hbm.at[c, pl.ds(row0, RPT)])


def _combine_body(p0_ref, p1_ref, emb_ref, g_ref, bt_ref, out_ref):
    h = p0_ref[...] + p1_ref[...] + emb_ref[...]
    mu = jnp.mean(h, axis=1, keepdims=True)
    dlt = h - mu
    var = jnp.mean(dlt * dlt, axis=1, keepdims=True)
    out_ref[...] = dlt * lax.rsqrt(var + 1e-5) * g_ref[...] + bt_ref[...]


def _combine(p0, p1, emb, gamma, beta):
    BN = 1000
    return pl.pallas_call(
        _combine_body,
        grid=(N // BN,),
        in_specs=[
            pl.BlockSpec((BN, D), lambda i: (i, 0)),
            pl.BlockSpec((BN, D), lambda i: (i, 0)),
            pl.BlockSpec((BN, D), lambda i: (i, 0)),
            pl.BlockSpec((1, D), lambda i: (0, 0)),
            pl.BlockSpec((1, D), lambda i: (0, 0)),
        ],
        out_specs=pl.BlockSpec((BN, D), lambda i: (i, 0)),
        out_shape=jax.ShapeDtypeStruct((N, D), jnp.float32),
    )(p0, p1, emb, gamma.reshape(1, D), beta.reshape(1, D))


def kernel(entity_emb, edge_index, edge_type, W, b, gamma, beta):
    src = edge_index[0].astype(jnp.int32)
    dst = edge_index[1].astype(jnp.int32)
    et = edge_type.astype(jnp.int32)

    npad = EPAD - E
    src_p = jnp.concatenate([src, jnp.zeros((npad,), jnp.int32)])
    et_p = jnp.concatenate([et, jnp.zeros((npad,), jnp.int32)])
    dst_p = jnp.concatenate([dst, jnp.full((npad,), DUMMY, jnp.int32)])

    t_table = _transform(entity_emb, W, b)
    gidx = _make_gidx(et_p, src_p)
    partials = _sc_scatter_kernel()(gidx, dst_p, t_table)
    return _combine(partials[0, :N], partials[1, :N], entity_emb, gamma, beta)


# speedup vs baseline: 17.6854x; 1.2885x over previous
import functools

import jax
import jax.numpy as jnp
from jax import lax
from jax.experimental import pallas as pl
from jax.experimental.pallas import tpu as pltpu
from jax.experimental.pallas import tpu_sc as plsc

N = 10000
D = 128
R = 8
E = 320000

CHUNK = 128
CPW = 79
NW = 32
EPAD = NW * CPW * CHUNK
DBITS = 14
NPAD = 10112
RPT = NPAD // 16
DUMMY = N


def _transform_body(emb_ref, w_ref, b_ref, t_ref):
    x = emb_ref[...]
    w = w_ref[0]
    y = lax.dot_general(x, w, (((1,), (1,)), ((), ())),
                        preferred_element_type=jnp.float32)
    t_ref[0] = y + b_ref[0]


def _transform(emb, W, b):
    BN = 1000
    out = pl.pallas_call(
        _transform_body,
        grid=(R, N // BN),
        in_specs=[
            pl.BlockSpec((BN, D), lambda r, i: (i, 0)),
            pl.BlockSpec((1, D, D), lambda r, i: (r, 0, 0)),
            pl.BlockSpec((1, 1, D), lambda r, i: (r, 0, 0)),
        ],
        out_specs=pl.BlockSpec((1, BN, D), lambda r, i: (r, i, 0)),
        out_shape=jax.ShapeDtypeStruct((R, N, D), jnp.float32),
    )(emb, W, b.reshape(R, 1, D))
    return out.reshape(R * N, D)


def _pidx_body(et_ref, src_ref, dst_ref, p_ref):
    gidx = et_ref[...] * N + src_ref[...]
    p_ref[...] = gidx * (1 << DBITS) + dst_ref[...]


def _make_pidx(et_p, src_p, dst_p):
    rows = EPAD // 128
    BR = rows // 2
    out = pl.pallas_call(
        _pidx_body,
        grid=(2,),
        in_specs=[
            pl.BlockSpec((BR, 128), lambda i: (i, 0)),
            pl.BlockSpec((BR, 128), lambda i: (i, 0)),
            pl.BlockSpec((BR, 128), lambda i: (i, 0)),
        ],
        out_specs=pl.BlockSpec((BR, 128), lambda i: (i, 0)),
        out_shape=jax.ShapeDtypeStruct((rows, 128), jnp.int32),
    )(et_p.reshape(rows, 128), src_p.reshape(rows, 128), dst_p.reshape(rows, 128))
    return out.reshape(EPAD)


@functools.cache
def _sc_scatter_kernel():
    mesh = plsc.VectorSubcoreMesh(core_axis_name="c", subcore_axis_name="s")
    return functools.partial(
        pl.kernel,
        mesh=mesh,
        out_type=jax.ShapeDtypeStruct((2, NPAD, D), jnp.float32),
        scratch_types=[
            pltpu.VMEM((CPW, CHUNK), jnp.int32),
            pltpu.VMEM((2, CHUNK), jnp.int32),
            pltpu.VMEM((2, CHUNK), jnp.int32),
            pltpu.VMEM((2, CHUNK, D), jnp.float32),
            pltpu.VMEM_SHARED((NPAD, D), jnp.float32),
            pltpu.SemaphoreType.DMA,
            pltpu.SemaphoreType.DMA,
        ],
    )(_sc_scatter_body)


def _sc_scatter_body(pidx_hbm, t_hbm, out_hbm, pbuf, gch, dch, rows, acc,
                     sem0, sem1):
    c = lax.axis_index("c")
    s = lax.axis_index("s")
    wid = s * 2 + c
    row0 = s * RPT

    pltpu.async_copy(pidx_hbm.at[wid], pbuf, sem0)

    zero16 = jnp.zeros((16,), jnp.float32)

    def zbody(i, carry):
        for j in range(8):
            rows[0, i, pl.ds(j * 16, 16)] = zero16
        return carry

    lax.fori_loop(0, CHUNK, zbody, 0)
    nfull, rem = RPT // CHUNK, RPT % CHUNK
    for t in range(nfull):
        pltpu.sync_copy(rows.at[0], acc.at[pl.ds(row0 + t * CHUNK, CHUNK)])
    if rem:
        pltpu.sync_copy(rows.at[0, pl.ds(0, rem)],
                        acc.at[pl.ds(row0 + nfull * CHUNK, rem)])
    pltpu.make_async_copy(pidx_hbm.at[wid], pbuf, sem0).wait()
    plsc.subcore_barrier()

    dmask = jnp.full((16,), (1 << DBITS) - 1, jnp.int32)

    def unpack(j, b):
        for i in range(CHUNK // 16):
            sl = pl.ds(i * 16, 16)
            v = pbuf[j, sl]
            gch[b, sl] = lax.shift_right_logical(v, DBITS)
            dch[b, sl] = lax.bitwise_and(v, dmask)

    def g_start(b, sem):
        pltpu.async_copy(t_hbm.at[gch.at[b]], rows.at[b], sem)

    def g_wait(b, sem):
        pltpu.make_async_copy(t_hbm.at[gch.at[b]], rows.at[b], sem).wait()

    def s_add(b):
        pltpu.sync_copy(rows.at[b], acc.at[dch.at[b]], add=True)

    unpack(0, 0)
    g_start(0, sem0)

    def body(k, carry):
        c0 = 2 * k
        unpack(c0 + 1, 1)
        g_start(1, sem1)
        g_wait(0, sem0)
        s_add(0)
        unpack(c0 + 2, 0)
        g_start(0, sem0)
        g_wait(1, sem1)
        s_add(1)
        return carry

    lax.fori_loop(0, (CPW - 1) // 2, body, 0)
    g_wait(0, sem0)
    s_add(0)
    plsc.subcore_barrier()

    pltpu.sync_copy(acc.at[pl.ds(row0, RPT)],
                    out_hbm.at[c, pl.ds(row0, RPT)])


def _combine_body(p0_ref, p1_ref, emb_ref, g_ref, bt_ref, out_ref):
    h = p0_ref[...] + p1_ref[...] + emb_ref[...]
    mu = jnp.mean(h, axis=1, keepdims=True)
    dlt = h - mu
    var = jnp.mean(dlt * dlt, axis=1, keepdims=True)
    out_ref[...] = dlt * lax.rsqrt(var + 1e-5) * g_ref[...] + bt_ref[...]


def _combine(p0, p1, emb, gamma, beta):
    BN = 1000
    return pl.pallas_call(
        _combine_body,
        grid=(N // BN,),
        in_specs=[
            pl.BlockSpec((BN, D), lambda i: (i, 0)),
            pl.BlockSpec((BN, D), lambda i: (i, 0)),
            pl.BlockSpec((BN, D), lambda i: (i, 0)),
            pl.BlockSpec((1, D), lambda i: (0, 0)),
            pl.BlockSpec((1, D), lambda i: (0, 0)),
        ],
        out_specs=pl.BlockSpec((BN, D), lambda i: (i, 0)),
        out_shape=jax.ShapeDtypeStruct((N, D), jnp.float32),
    )(p0, p1, emb, gamma.reshape(1, D), beta.reshape(1, D))


def kernel(entity_emb, edge_index, edge_type, W, b, gamma, beta):
    src = edge_index[0].astype(jnp.int32)
    dst = edge_index[1].astype(jnp.int32)
    et = edge_type.astype(jnp.int32)

    npad = EPAD - E
    src_p = jnp.concatenate([src, jnp.zeros((npad,), jnp.int32)])
    et_p = jnp.concatenate([et, jnp.zeros((npad,), jnp.int32)])
    dst_p = jnp.concatenate([dst, jnp.full((npad,), DUMMY, jnp.int32)])

    t_table = _transform(entity_emb, W, b)
    pidx = _make_pidx(et_p, src_p, dst_p)
    partials = _sc_scatter_kernel()(pidx.reshape(NW, CPW, CHUNK),
                                    t_table)
    return _combine(partials[0, :N], partials[1, :N], entity_emb, gamma, beta)
